# baseline (speedup 1.0000x reference)
import functools

import jax
import jax.numpy as jnp
from jax import lax
from jax.experimental import pallas as pl
from jax.experimental.pallas import tpu as pltpu
from jax.experimental.pallas import tpu_sc as plsc

E = 8
TOPK = 2
H = 1024
DFF = 512
NSH = 2

N = 2048
NPAIR = N * TOPK
BLK = 512
NB = NPAIR // BLK + (E - 1)
PAD = NB * BLK

SC_NC = 2
SC_NS = 16
SC_NW = SC_NC * SC_NS
TOK_PER_W = N // SC_NW


def _gate_body(x_ref, gw_ref, pos1_ref, pos2_ref, w1_ref, w2_ref,
               be_ref, nbc_ref):
    x = x_ref[...]
    logits = lax.dot_general(x, gw_ref[...], (((1,), (1,)), ((), ())),
                             preferred_element_type=jnp.float32)
    m = jnp.max(logits, axis=-1, keepdims=True)
    p = jnp.exp(logits - m)
    scores = p / jnp.sum(p, axis=-1, keepdims=True)
    lanes = lax.broadcasted_iota(jnp.int32, scores.shape, 1)
    i1 = jnp.argmax(scores, axis=-1)
    s1 = jnp.max(scores, axis=-1)
    masked = jnp.where(lanes == i1[:, None], -jnp.inf, scores)
    i2 = jnp.argmax(masked, axis=-1)
    s2 = jnp.max(masked, axis=-1)
    den = s1 + s2 + 1e-20
    w1_ref[...] = s1 / den
    w2_ref[...] = s2 / den

    oh1 = (lanes == i1[:, None]).astype(jnp.int32)
    oh2 = (lanes == i2[:, None]).astype(jnp.int32)
    oh = oh1 + oh2
    inc = oh
    shift = 1
    while shift < N:
        inc = inc + jnp.pad(inc, ((shift, 0), (0, 0)))[:N, :]
        shift *= 2
    excl = inc - oh
    counts = inc[N - 1:N, :]
    nb = (counts + BLK - 1) // BLK
    nbc = nb
    shift = 1
    while shift < E:
        nbc = nbc + jnp.pad(nbc, ((0, 0), (shift, 0)))[:, :E]
        shift *= 2
    seg = (nbc - nb) * BLK
    pos_base = seg + excl
    pos1_ref[...] = jnp.sum(oh1 * pos_base, axis=1)
    pos2_ref[...] = jnp.sum(oh2 * (pos_base + oh1), axis=1)
    nbc_ref[...] = nbc.reshape(E)

    bb = lax.broadcasted_iota(jnp.int32, (NB, E), 0)
    be = jnp.sum((nbc <= bb).astype(jnp.int32), axis=1)
    be_ref[...] = jnp.clip(be, 0, E - 1)


def _gate_call(x, gate_w):
    return pl.pallas_call(
        _gate_body,
        in_specs=[
            pl.BlockSpec((N, H), lambda: (0, 0)),
            pl.BlockSpec((E, H), lambda: (0, 0)),
        ],
        out_specs=[
            pl.BlockSpec((N,), lambda: (0,)),
            pl.BlockSpec((N,), lambda: (0,)),
            pl.BlockSpec((N,), lambda: (0,)),
            pl.BlockSpec((N,), lambda: (0,)),
            pl.BlockSpec((NB,), lambda: (0,)),
            pl.BlockSpec((E,), lambda: (0,)),
        ],
        out_shape=[
            jax.ShapeDtypeStruct((N,), jnp.int32),
            jax.ShapeDtypeStruct((N,), jnp.int32),
            jax.ShapeDtypeStruct((N,), jnp.float32),
            jax.ShapeDtypeStruct((N,), jnp.float32),
            jax.ShapeDtypeStruct((NB,), jnp.int32),
            jax.ShapeDtypeStruct((E,), jnp.int32),
        ],
    )(x, gate_w)


def _sc_dispatch(x, pos12):
    def body(x_hbm, pos_hbm, xs_hbm, idx_v, row_v, sem):
        wid = lax.axis_index("s") * SC_NC + lax.axis_index("c")
        base = wid * TOK_PER_W
        pltpu.sync_copy(x_hbm.at[pl.ds(base, TOK_PER_W)], row_v)
        for k in range(TOPK):
            pltpu.sync_copy(pos_hbm.at[wid, k], idx_v)
            pltpu.async_copy(row_v, xs_hbm.at[idx_v], sem).wait()

    mesh = plsc.VectorSubcoreMesh(core_axis_name="c", subcore_axis_name="s",
                                  num_cores=SC_NC, num_subcores=SC_NS)
    fn = pl.kernel(body,
                   out_type=jax.ShapeDtypeStruct((PAD, H), jnp.float32),
                   mesh=mesh,
                   scratch_types=[
                       pltpu.VMEM((TOK_PER_W,), jnp.int32),
                       pltpu.VMEM((TOK_PER_W, H), jnp.float32),
                       pltpu.SemaphoreType.DMA,
                   ])
    return fn(x, pos12)


def _sc_gather(table, idx):
    rows, h = int(idx.shape[0]), int(table.shape[1])
    per = rows // SC_NW
    assert per * SC_NW == rows and per % 64 == 0
    chunks = per // 64

    def body(table_hbm, idx_hbm, out_hbm, iv, rv, sem):
        wid = lax.axis_index("s") * SC_NC + lax.axis_index("c")
        base = wid * per
        for c in range(chunks):
            pltpu.sync_copy(idx_hbm.at[pl.ds(base + c * 64, 64)], iv)
            pltpu.async_copy(table_hbm.at[iv], rv, sem).wait()
            pltpu.sync_copy(rv, out_hbm.at[pl.ds(base + c * 64, 64)])

    mesh = plsc.VectorSubcoreMesh(core_axis_name="c", subcore_axis_name="s",
                                  num_cores=SC_NC, num_subcores=SC_NS)
    fn = pl.kernel(body,
                   out_type=jax.ShapeDtypeStruct((rows, h), jnp.float32),
                   mesh=mesh,
                   scratch_types=[
                       pltpu.VMEM((64,), jnp.int32),
                       pltpu.VMEM((64, h), jnp.float32),
                       pltpu.SemaphoreType.DMA,
                   ])
    return fn(table, idx)


def _routed_body(total_ref, be_ref, bx_ref, xs_ref, wg_ref, wu_ref, wd_ref,
                 ys_ref):
    i = pl.program_id(0)

    @pl.when(i < total_ref[0])
    def _():
        x = xs_ref[...].astype(jnp.bfloat16)
        hg = lax.dot_general(x, wg_ref[0], (((1,), (0,)), ((), ())),
                             preferred_element_type=jnp.float32)
        hu = lax.dot_general(x, wu_ref[0], (((1,), (0,)), ((), ())),
                             preferred_element_type=jnp.float32)
        act = ((hg * lax.logistic(hg)) * hu).astype(jnp.bfloat16)
        ys_ref[...] = lax.dot_general(act, wd_ref[0], (((1,), (0,)), ((), ())),
                                      preferred_element_type=jnp.float32)


def _routed_call(total, block_expert, bx, xs, w_gate, w_up, w_down):
    grid_spec = pltpu.PrefetchScalarGridSpec(
        num_scalar_prefetch=3,
        grid=(NB,),
        in_specs=[
            pl.BlockSpec((BLK, H), lambda i, t, be, bx: (bx[i], 0)),
            pl.BlockSpec((1, H, DFF), lambda i, t, be, bx: (be[i], 0, 0)),
            pl.BlockSpec((1, H, DFF), lambda i, t, be, bx: (be[i], 0, 0)),
            pl.BlockSpec((1, DFF, H), lambda i, t, be, bx: (be[i], 0, 0)),
        ],
        out_specs=pl.BlockSpec((BLK, H), lambda i, t, be, bx: (bx[i], 0)),
    )
    return pl.pallas_call(
        _routed_body,
        grid_spec=grid_spec,
        out_shape=jax.ShapeDtypeStruct((PAD, H), jnp.float32),
        compiler_params=pltpu.CompilerParams(
            dimension_semantics=("arbitrary",)),
    )(total, block_expert, bx, xs, w_gate, w_up, w_down)


def _shared_body(x_ref, swg_ref, swu_ref, swd_ref, ysh_ref):
    x = x_ref[...].astype(jnp.bfloat16)
    hg = lax.dot_general(x, swg_ref[...], (((1,), (0,)), ((), ())),
                         preferred_element_type=jnp.float32)
    hu = lax.dot_general(x, swu_ref[...], (((1,), (0,)), ((), ())),
                         preferred_element_type=jnp.float32)
    act = ((hg * lax.logistic(hg)) * hu).astype(jnp.bfloat16)
    ysh_ref[...] = lax.dot_general(act, swd_ref[...], (((1,), (0,)), ((), ())),
                                   preferred_element_type=jnp.float32)


def _shared_call(x, sw_gate, sw_up, sw_down):
    bt = 512
    tb = N // bt
    return pl.pallas_call(
        _shared_body,
        grid=(tb,),
        in_specs=[
            pl.BlockSpec((bt, H), lambda t: (t, 0)),
            pl.BlockSpec((H, DFF * NSH), lambda t: (0, 0)),
            pl.BlockSpec((H, DFF * NSH), lambda t: (0, 0)),
            pl.BlockSpec((DFF * NSH, H), lambda t: (0, 0)),
        ],
        out_specs=pl.BlockSpec((bt, H), lambda t: (t, 0)),
        out_shape=jax.ShapeDtypeStruct((N, H), jnp.float32),
        compiler_params=pltpu.CompilerParams(
            dimension_semantics=("parallel",)),
    )(x, sw_gate, sw_up, sw_down)


def _combine_body(ysh_ref, y0_ref, y1_ref, w1_ref, w2_ref, y_ref):
    y_ref[...] = (ysh_ref[...] + w1_ref[...] * y0_ref[...]
                  + w2_ref[...] * y1_ref[...])


def _combine_call(ysh, y01, w1, w2):
    bt = 512
    tb = N // bt
    return pl.pallas_call(
        _combine_body,
        grid=(tb,),
        in_specs=[
            pl.BlockSpec((bt, H), lambda t: (t, 0)),
            pl.BlockSpec((bt, H), lambda t: (t, 0)),
            pl.BlockSpec((bt, H), lambda t: (t + tb, 0)),
            pl.BlockSpec((bt, 1), lambda t: (t, 0)),
            pl.BlockSpec((bt, 1), lambda t: (t, 0)),
        ],
        out_specs=pl.BlockSpec((bt, H), lambda t: (t, 0)),
        out_shape=jax.ShapeDtypeStruct((N, H), jnp.float32),
        compiler_params=pltpu.CompilerParams(
            dimension_semantics=("parallel",)),
    )(ysh, y01, y01, w1, w2)


def kernel(hidden_states, gate_w, w_gate, w_up, w_down, sw_gate, sw_up, sw_down):
    b, s, h = hidden_states.shape
    x = hidden_states.reshape(-1, h)

    pos1, pos2, w1, w2, block_expert, nbc = _gate_call(x, gate_w)
    total = nbc[E - 1:E]
    bx = jnp.minimum(jnp.arange(NB, dtype=jnp.int32), total - 1)
    pos12 = jnp.stack([pos1.reshape(SC_NW, TOK_PER_W),
                       pos2.reshape(SC_NW, TOK_PER_W)], axis=1)

    xs = _sc_dispatch(x, pos12)

    ys = _routed_call(total, block_expert, bx, xs,
                      w_gate.astype(jnp.bfloat16),
                      w_up.astype(jnp.bfloat16),
                      w_down.astype(jnp.bfloat16))

    pos01 = jnp.concatenate([pos1, pos2]).astype(jnp.int32)
    y01 = _sc_gather(ys, pos01)

    ysh = _shared_call(x, sw_gate.astype(jnp.bfloat16),
                       sw_up.astype(jnp.bfloat16),
                       sw_down.astype(jnp.bfloat16))

    y = _combine_call(ysh, y01, w1[:, None], w2[:, None])
    return y.reshape(b, s, h)


# speedup vs baseline: 1.2184x; 1.2184x over previous
import functools

import jax
import jax.numpy as jnp
from jax import lax
from jax.experimental import pallas as pl
from jax.experimental.pallas import tpu as pltpu
from jax.experimental.pallas import tpu_sc as plsc

E = 8
TOPK = 2
H = 1024
DFF = 512
NSH = 2

N = 2048
NPAIR = N * TOPK
BLK = 512
NB = NPAIR // BLK + (E - 1)
PAD = NB * BLK

SC_NC = 2
SC_NS = 16
SC_NW = SC_NC * SC_NS
TOK_PER_W = N // SC_NW


def _gate_body(x_ref, gw_ref, pos1_ref, pos2_ref, w1_ref, w2_ref,
               be_ref, nbc_ref):
    x = x_ref[...]
    logits = lax.dot_general(x, gw_ref[...], (((1,), (1,)), ((), ())),
                             preferred_element_type=jnp.float32)
    m = jnp.max(logits, axis=-1, keepdims=True)
    p = jnp.exp(logits - m)
    scores = p / jnp.sum(p, axis=-1, keepdims=True)
    lanes = lax.broadcasted_iota(jnp.int32, scores.shape, 1)
    i1 = jnp.argmax(scores, axis=-1)
    s1 = jnp.max(scores, axis=-1)
    masked = jnp.where(lanes == i1[:, None], -jnp.inf, scores)
    i2 = jnp.argmax(masked, axis=-1)
    s2 = jnp.max(masked, axis=-1)
    den = s1 + s2 + 1e-20
    w1_ref[...] = s1 / den
    w2_ref[...] = s2 / den

    oh1 = (lanes == i1[:, None]).astype(jnp.int32)
    oh2 = (lanes == i2[:, None]).astype(jnp.int32)
    oh = oh1 + oh2
    inc = oh
    shift = 1
    while shift < N:
        inc = inc + jnp.pad(inc, ((shift, 0), (0, 0)))[:N, :]
        shift *= 2
    excl = inc - oh
    counts = inc[N - 1:N, :]
    nb = (counts + BLK - 1) // BLK
    nbc = nb
    shift = 1
    while shift < E:
        nbc = nbc + jnp.pad(nbc, ((0, 0), (shift, 0)))[:, :E]
        shift *= 2
    seg = (nbc - nb) * BLK
    pos_base = seg + excl
    pos1_ref[...] = jnp.sum(oh1 * pos_base, axis=1)
    pos2_ref[...] = jnp.sum(oh2 * (pos_base + oh1), axis=1)
    nbc_ref[...] = nbc.reshape(E)

    bb = lax.broadcasted_iota(jnp.int32, (NB, E), 0)
    be = jnp.sum((nbc <= bb).astype(jnp.int32), axis=1)
    be_ref[...] = jnp.clip(be, 0, E - 1)


def _gate_call(x, gate_w):
    return pl.pallas_call(
        _gate_body,
        in_specs=[
            pl.BlockSpec((N, H), lambda: (0, 0)),
            pl.BlockSpec((E, H), lambda: (0, 0)),
        ],
        out_specs=[
            pl.BlockSpec((N,), lambda: (0,)),
            pl.BlockSpec((N,), lambda: (0,)),
            pl.BlockSpec((N,), lambda: (0,)),
            pl.BlockSpec((N,), lambda: (0,)),
            pl.BlockSpec((NB,), lambda: (0,)),
            pl.BlockSpec((E,), lambda: (0,)),
        ],
        out_shape=[
            jax.ShapeDtypeStruct((N,), jnp.int32),
            jax.ShapeDtypeStruct((N,), jnp.int32),
            jax.ShapeDtypeStruct((N,), jnp.float32),
            jax.ShapeDtypeStruct((N,), jnp.float32),
            jax.ShapeDtypeStruct((NB,), jnp.int32),
            jax.ShapeDtypeStruct((E,), jnp.int32),
        ],
    )(x, gate_w)


def _sc_dispatch(x, pos12):
    def body(x_hbm, pos_hbm, xs_hbm, idx_v, row_v, sem):
        wid = lax.axis_index("s") * SC_NC + lax.axis_index("c")
        base = wid * TOK_PER_W
        pltpu.sync_copy(x_hbm.at[pl.ds(base, TOK_PER_W)], row_v)
        for k in range(TOPK):
            pltpu.sync_copy(pos_hbm.at[wid, k], idx_v)
            pltpu.async_copy(row_v, xs_hbm.at[idx_v], sem).wait()

    mesh = plsc.VectorSubcoreMesh(core_axis_name="c", subcore_axis_name="s",
                                  num_cores=SC_NC, num_subcores=SC_NS)
    fn = pl.kernel(body,
                   out_type=jax.ShapeDtypeStruct((PAD, H), jnp.float32),
                   mesh=mesh,
                   scratch_types=[
                       pltpu.VMEM((TOK_PER_W,), jnp.int32),
                       pltpu.VMEM((TOK_PER_W, H), jnp.float32),
                       pltpu.SemaphoreType.DMA,
                   ])
    return fn(x, pos12)


def _sc_gather(table, idx):
    rows, h = int(idx.shape[0]), int(table.shape[1])
    per = rows // SC_NW
    assert per * SC_NW == rows and per % 64 == 0
    chunks = per // 64

    def body(table_hbm, idx_hbm, out_hbm, iv, rv, sem):
        wid = lax.axis_index("s") * SC_NC + lax.axis_index("c")
        base = wid * per
        for c in range(chunks):
            pltpu.sync_copy(idx_hbm.at[pl.ds(base + c * 64, 64)], iv)
            pltpu.async_copy(table_hbm.at[iv], rv, sem).wait()
            pltpu.sync_copy(rv, out_hbm.at[pl.ds(base + c * 64, 64)])

    mesh = plsc.VectorSubcoreMesh(core_axis_name="c", subcore_axis_name="s",
                                  num_cores=SC_NC, num_subcores=SC_NS)
    fn = pl.kernel(body,
                   out_type=jax.ShapeDtypeStruct((rows, h), jnp.float32),
                   mesh=mesh,
                   scratch_types=[
                       pltpu.VMEM((64,), jnp.int32),
                       pltpu.VMEM((64, h), jnp.float32),
                       pltpu.SemaphoreType.DMA,
                   ])
    return fn(table, idx)


def _routed_body(total_ref, be_ref, bx_ref, xs_ref, wg_ref, wu_ref, wd_ref,
                 ys_ref):
    i = pl.program_id(0)

    @pl.when(i < total_ref[0])
    def _():
        x = xs_ref[...]
        hg = lax.dot_general(x, wg_ref[0], (((1,), (0,)), ((), ())),
                             preferred_element_type=jnp.float32)
        hu = lax.dot_general(x, wu_ref[0], (((1,), (0,)), ((), ())),
                             preferred_element_type=jnp.float32)
        act = (hg * lax.logistic(hg)) * hu
        ys_ref[...] = lax.dot_general(act, wd_ref[0], (((1,), (0,)), ((), ())),
                                      preferred_element_type=jnp.float32)


def _routed_call(total, block_expert, bx, xs, w_gate, w_up, w_down):
    grid_spec = pltpu.PrefetchScalarGridSpec(
        num_scalar_prefetch=3,
        grid=(NB,),
        in_specs=[
            pl.BlockSpec((BLK, H), lambda i, t, be, bx: (bx[i], 0)),
            pl.BlockSpec((1, H, DFF), lambda i, t, be, bx: (be[i], 0, 0)),
            pl.BlockSpec((1, H, DFF), lambda i, t, be, bx: (be[i], 0, 0)),
            pl.BlockSpec((1, DFF, H), lambda i, t, be, bx: (be[i], 0, 0)),
        ],
        out_specs=pl.BlockSpec((BLK, H), lambda i, t, be, bx: (bx[i], 0)),
    )
    return pl.pallas_call(
        _routed_body,
        grid_spec=grid_spec,
        out_shape=jax.ShapeDtypeStruct((PAD, H), jnp.float32),
        compiler_params=pltpu.CompilerParams(
            dimension_semantics=("arbitrary",)),
    )(total, block_expert, bx, xs, w_gate, w_up, w_down)


def _shared_body(x_ref, swg_ref, swu_ref, swd_ref, ysh_ref):
    x = x_ref[...]
    hg = lax.dot_general(x, swg_ref[...], (((1,), (0,)), ((), ())),
                         preferred_element_type=jnp.float32)
    hu = lax.dot_general(x, swu_ref[...], (((1,), (0,)), ((), ())),
                         preferred_element_type=jnp.float32)
    act = (hg * lax.logistic(hg)) * hu
    ysh_ref[...] = lax.dot_general(act, swd_ref[...], (((1,), (0,)), ((), ())),
                                   preferred_element_type=jnp.float32)


def _shared_call(x, sw_gate, sw_up, sw_down):
    bt = 512
    tb = N // bt
    return pl.pallas_call(
        _shared_body,
        grid=(tb,),
        in_specs=[
            pl.BlockSpec((bt, H), lambda t: (t, 0)),
            pl.BlockSpec((H, DFF * NSH), lambda t: (0, 0)),
            pl.BlockSpec((H, DFF * NSH), lambda t: (0, 0)),
            pl.BlockSpec((DFF * NSH, H), lambda t: (0, 0)),
        ],
        out_specs=pl.BlockSpec((bt, H), lambda t: (t, 0)),
        out_shape=jax.ShapeDtypeStruct((N, H), jnp.float32),
        compiler_params=pltpu.CompilerParams(
            dimension_semantics=("parallel",)),
    )(x, sw_gate, sw_up, sw_down)


def _combine_body(ysh_ref, y0_ref, y1_ref, w1_ref, w2_ref, y_ref):
    y_ref[...] = (ysh_ref[...] + w1_ref[...] * y0_ref[...]
                  + w2_ref[...] * y1_ref[...])


def _combine_call(ysh, y01, w1, w2):
    bt = 512
    tb = N // bt
    return pl.pallas_call(
        _combine_body,
        grid=(tb,),
        in_specs=[
            pl.BlockSpec((bt, H), lambda t: (t, 0)),
            pl.BlockSpec((bt, H), lambda t: (t, 0)),
            pl.BlockSpec((bt, H), lambda t: (t + tb, 0)),
            pl.BlockSpec((bt, 1), lambda t: (t, 0)),
            pl.BlockSpec((bt, 1), lambda t: (t, 0)),
        ],
        out_specs=pl.BlockSpec((bt, H), lambda t: (t, 0)),
        out_shape=jax.ShapeDtypeStruct((N, H), jnp.float32),
        compiler_params=pltpu.CompilerParams(
            dimension_semantics=("parallel",)),
    )(ysh, y01, y01, w1, w2)


def kernel(hidden_states, gate_w, w_gate, w_up, w_down, sw_gate, sw_up, sw_down):
    b, s, h = hidden_states.shape
    x = hidden_states.reshape(-1, h)

    pos1, pos2, w1, w2, block_expert, nbc = _gate_call(x, gate_w)
    total = nbc[E - 1:E]
    bx = jnp.minimum(jnp.arange(NB, dtype=jnp.int32), total - 1)
    pos12 = jnp.stack([pos1.reshape(SC_NW, TOK_PER_W),
                       pos2.reshape(SC_NW, TOK_PER_W)], axis=1)

    xs = _sc_dispatch(x, pos12)

    ys = _routed_call(total, block_expert, bx, xs, w_gate, w_up, w_down)

    pos01 = jnp.concatenate([pos1, pos2]).astype(jnp.int32)
    y01 = _sc_gather(ys, pos01)

    ysh = _shared_call(x, sw_gate, sw_up, sw_down)

    y = _combine_call(ysh, y01, w1[:, None], w2[:, None])
    return y.reshape(b, s, h)


# speedup vs baseline: 1.2503x; 1.0262x over previous
import functools

import jax
import jax.numpy as jnp
from jax import lax
from jax.experimental import pallas as pl
from jax.experimental.pallas import tpu as pltpu
from jax.experimental.pallas import tpu_sc as plsc

E = 8
TOPK = 2
H = 1024
DFF = 512
NSH = 2

N = 2048
NPAIR = N * TOPK
BLK = 512
NB = NPAIR // BLK + (E - 1)
PAD = NB * BLK

SC_NC = 2
SC_NS = 16
SC_NW = SC_NC * SC_NS
TOK_PER_W = N // SC_NW


def _gate_body(x_ref, gw_ref, pos1_ref, pos2_ref, w1_ref, w2_ref,
               be_ref, nbc_ref, bx_ref):
    x = x_ref[...]
    logits = lax.dot_general(x, gw_ref[...], (((1,), (1,)), ((), ())),
                             preferred_element_type=jnp.float32)
    m = jnp.max(logits, axis=-1, keepdims=True)
    p = jnp.exp(logits - m)
    scores = p / jnp.sum(p, axis=-1, keepdims=True)
    lanes = lax.broadcasted_iota(jnp.int32, scores.shape, 1)
    i1 = jnp.argmax(scores, axis=-1)
    s1 = jnp.max(scores, axis=-1)
    masked = jnp.where(lanes == i1[:, None], -jnp.inf, scores)
    i2 = jnp.argmax(masked, axis=-1)
    s2 = jnp.max(masked, axis=-1)
    den = s1 + s2 + 1e-20
    w1_ref[...] = s1 / den
    w2_ref[...] = s2 / den

    oh1 = (lanes == i1[:, None]).astype(jnp.int32)
    oh2 = (lanes == i2[:, None]).astype(jnp.int32)
    oh = oh1 + oh2
    inc = oh
    shift = 1
    while shift < N:
        inc = inc + jnp.pad(inc, ((shift, 0), (0, 0)))[:N, :]
        shift *= 2
    excl = inc - oh
    counts = inc[N - 1:N, :]
    nb = (counts + BLK - 1) // BLK
    nbc = nb
    shift = 1
    while shift < E:
        nbc = nbc + jnp.pad(nbc, ((0, 0), (shift, 0)))[:, :E]
        shift *= 2
    seg = (nbc - nb) * BLK
    pos_base = seg + excl
    pos1_ref[...] = jnp.sum(oh1 * pos_base, axis=1)
    pos2_ref[...] = jnp.sum(oh2 * (pos_base + oh1), axis=1)
    nbc_ref[...] = nbc.reshape(E)

    bb = lax.broadcasted_iota(jnp.int32, (NB, E), 0)
    be = jnp.sum((nbc <= bb).astype(jnp.int32), axis=1)
    be_ref[...] = jnp.clip(be, 0, E - 1)
    total = jnp.sum(jnp.where(
        lax.broadcasted_iota(jnp.int32, (1, E), 1) == E - 1, nbc, 0))
    bx_ref[...] = jnp.minimum(
        lax.broadcasted_iota(jnp.int32, (NB, E), 0)[:, 0], total - 1)


def _gate_call(x, gate_w):
    return pl.pallas_call(
        _gate_body,
        in_specs=[
            pl.BlockSpec((N, H), lambda: (0, 0)),
            pl.BlockSpec((E, H), lambda: (0, 0)),
        ],
        out_specs=[
            pl.BlockSpec((N,), lambda: (0,)),
            pl.BlockSpec((N,), lambda: (0,)),
            pl.BlockSpec((N,), lambda: (0,)),
            pl.BlockSpec((N,), lambda: (0,)),
            pl.BlockSpec((NB,), lambda: (0,)),
            pl.BlockSpec((E,), lambda: (0,)),
            pl.BlockSpec((NB,), lambda: (0,)),
        ],
        out_shape=[
            jax.ShapeDtypeStruct((N,), jnp.int32),
            jax.ShapeDtypeStruct((N,), jnp.int32),
            jax.ShapeDtypeStruct((N,), jnp.float32),
            jax.ShapeDtypeStruct((N,), jnp.float32),
            jax.ShapeDtypeStruct((NB,), jnp.int32),
            jax.ShapeDtypeStruct((E,), jnp.int32),
            jax.ShapeDtypeStruct((NB,), jnp.int32),
        ],
    )(x, gate_w)


def _sc_dispatch(x, pos1, pos2):
    def body(x_hbm, pos1_hbm, pos2_hbm, xs_hbm, idx_v, row_v, sem):
        wid = lax.axis_index("s") * SC_NC + lax.axis_index("c")
        base = wid * TOK_PER_W
        pltpu.sync_copy(x_hbm.at[pl.ds(base, TOK_PER_W)], row_v)
        for pos_hbm in (pos1_hbm, pos2_hbm):
            pltpu.sync_copy(pos_hbm.at[pl.ds(base, TOK_PER_W)], idx_v)
            pltpu.async_copy(row_v, xs_hbm.at[idx_v], sem).wait()

    mesh = plsc.VectorSubcoreMesh(core_axis_name="c", subcore_axis_name="s",
                                  num_cores=SC_NC, num_subcores=SC_NS)
    fn = pl.kernel(body,
                   out_type=jax.ShapeDtypeStruct((PAD, H), jnp.float32),
                   mesh=mesh,
                   scratch_types=[
                       pltpu.VMEM((TOK_PER_W,), jnp.int32),
                       pltpu.VMEM((TOK_PER_W, H), jnp.float32),
                       pltpu.SemaphoreType.DMA,
                   ])
    return fn(x, pos1, pos2)


def _sc_gather(table, pos1, pos2):
    h = int(table.shape[1])

    def body(table_hbm, pos1_hbm, pos2_hbm, out_hbm, iv, rv, sem):
        wid = lax.axis_index("s") * SC_NC + lax.axis_index("c")
        base = wid * TOK_PER_W
        for k, pos_hbm in enumerate((pos1_hbm, pos2_hbm)):
            pltpu.sync_copy(pos_hbm.at[pl.ds(base, TOK_PER_W)], iv)
            pltpu.async_copy(table_hbm.at[iv], rv, sem).wait()
            pltpu.sync_copy(rv, out_hbm.at[pl.ds(k * N + base, TOK_PER_W)])

    mesh = plsc.VectorSubcoreMesh(core_axis_name="c", subcore_axis_name="s",
                                  num_cores=SC_NC, num_subcores=SC_NS)
    fn = pl.kernel(body,
                   out_type=jax.ShapeDtypeStruct((TOPK * N, h), jnp.float32),
                   mesh=mesh,
                   scratch_types=[
                       pltpu.VMEM((TOK_PER_W,), jnp.int32),
                       pltpu.VMEM((TOK_PER_W, h), jnp.float32),
                       pltpu.SemaphoreType.DMA,
                   ])
    return fn(table, pos1, pos2)


def _routed_body(nbc_ref, be_ref, bx_ref, xs_ref, wg_ref, wu_ref, wd_ref,
                 ys_ref):
    i = pl.program_id(0)

    @pl.when(i < nbc_ref[E - 1])
    def _():
        x = xs_ref[...]
        hg = lax.dot_general(x, wg_ref[0], (((1,), (0,)), ((), ())),
                             preferred_element_type=jnp.float32)
        hu = lax.dot_general(x, wu_ref[0], (((1,), (0,)), ((), ())),
                             preferred_element_type=jnp.float32)
        act = (hg * lax.logistic(hg)) * hu
        ys_ref[...] = lax.dot_general(act, wd_ref[0], (((1,), (0,)), ((), ())),
                                      preferred_element_type=jnp.float32)


def _routed_call(nbc, block_expert, bx, xs, w_gate, w_up, w_down):
    grid_spec = pltpu.PrefetchScalarGridSpec(
        num_scalar_prefetch=3,
        grid=(NB,),
        in_specs=[
            pl.BlockSpec((BLK, H), lambda i, t, be, bx: (bx[i], 0)),
            pl.BlockSpec((1, H, DFF), lambda i, t, be, bx: (be[i], 0, 0)),
            pl.BlockSpec((1, H, DFF), lambda i, t, be, bx: (be[i], 0, 0)),
            pl.BlockSpec((1, DFF, H), lambda i, t, be, bx: (be[i], 0, 0)),
        ],
        out_specs=pl.BlockSpec((BLK, H), lambda i, t, be, bx: (bx[i], 0)),
    )
    return pl.pallas_call(
        _routed_body,
        grid_spec=grid_spec,
        out_shape=jax.ShapeDtypeStruct((PAD, H), jnp.float32),
        compiler_params=pltpu.CompilerParams(
            dimension_semantics=("arbitrary",)),
    )(nbc, block_expert, bx, xs, w_gate, w_up, w_down)


def _shared_body(x_ref, swg_ref, swu_ref, swd_ref, ysh_ref):
    x = x_ref[...]
    hg = lax.dot_general(x, swg_ref[...], (((1,), (0,)), ((), ())),
                         preferred_element_type=jnp.float32)
    hu = lax.dot_general(x, swu_ref[...], (((1,), (0,)), ((), ())),
                         preferred_element_type=jnp.float32)
    act = (hg * lax.logistic(hg)) * hu
    ysh_ref[...] = lax.dot_general(act, swd_ref[...], (((1,), (0,)), ((), ())),
                                   preferred_element_type=jnp.float32)


def _shared_call(x, sw_gate, sw_up, sw_down):
    bt = 512
    tb = N // bt
    return pl.pallas_call(
        _shared_body,
        grid=(tb,),
        in_specs=[
            pl.BlockSpec((bt, H), lambda t: (t, 0)),
            pl.BlockSpec((H, DFF * NSH), lambda t: (0, 0)),
            pl.BlockSpec((H, DFF * NSH), lambda t: (0, 0)),
            pl.BlockSpec((DFF * NSH, H), lambda t: (0, 0)),
        ],
        out_specs=pl.BlockSpec((bt, H), lambda t: (t, 0)),
        out_shape=jax.ShapeDtypeStruct((N, H), jnp.float32),
        compiler_params=pltpu.CompilerParams(
            dimension_semantics=("parallel",)),
    )(x, sw_gate, sw_up, sw_down)


def _combine_body(ysh_ref, y0_ref, y1_ref, w1_ref, w2_ref, y_ref):
    y_ref[...] = (ysh_ref[...] + w1_ref[...] * y0_ref[...]
                  + w2_ref[...] * y1_ref[...])


def _combine_call(ysh, y01, w1, w2):
    bt = 512
    tb = N // bt
    return pl.pallas_call(
        _combine_body,
        grid=(tb,),
        in_specs=[
            pl.BlockSpec((bt, H), lambda t: (t, 0)),
            pl.BlockSpec((bt, H), lambda t: (t, 0)),
            pl.BlockSpec((bt, H), lambda t: (t + tb, 0)),
            pl.BlockSpec((bt, 1), lambda t: (t, 0)),
            pl.BlockSpec((bt, 1), lambda t: (t, 0)),
        ],
        out_specs=pl.BlockSpec((bt, H), lambda t: (t, 0)),
        out_shape=jax.ShapeDtypeStruct((N, H), jnp.float32),
        compiler_params=pltpu.CompilerParams(
            dimension_semantics=("parallel",)),
    )(ysh, y01, y01, w1, w2)


def kernel(hidden_states, gate_w, w_gate, w_up, w_down, sw_gate, sw_up, sw_down):
    b, s, h = hidden_states.shape
    x = hidden_states.reshape(-1, h)

    pos1, pos2, w1, w2, block_expert, nbc, bx = _gate_call(x, gate_w)

    xs = _sc_dispatch(x, pos1, pos2)

    ys = _routed_call(nbc, block_expert, bx, xs, w_gate, w_up, w_down)

    y01 = _sc_gather(ys, pos1, pos2)

    ysh = _shared_call(x, sw_gate, sw_up, sw_down)

    y = _combine_call(ysh, y01, w1[:, None], w2[:, None])
    return y.reshape(b, s, h)


# speedup vs baseline: 1.2760x; 1.0205x over previous
import functools

import jax
import jax.numpy as jnp
from jax import lax
from jax.experimental import pallas as pl
from jax.experimental.pallas import tpu as pltpu
from jax.experimental.pallas import tpu_sc as plsc

E = 8
TOPK = 2
H = 1024
DFF = 512
NSH = 2

N = 2048
NPAIR = N * TOPK
BLK = 512
NB = NPAIR // BLK + (E - 1)
PAD = NB * BLK

SC_NC = 2
SC_NS = 16
SC_NW = SC_NC * SC_NS
TOK_PER_W = N // SC_NW


def _gate_body(x_ref, gw_ref, pos1_ref, pos2_ref, w1_ref, w2_ref,
               be_ref, nbc_ref, bx_ref):
    x = x_ref[...]
    logits = lax.dot_general(x, gw_ref[...], (((1,), (1,)), ((), ())),
                             preferred_element_type=jnp.float32)
    m = jnp.max(logits, axis=-1, keepdims=True)
    p = jnp.exp(logits - m)
    scores = p / jnp.sum(p, axis=-1, keepdims=True)
    lanes = lax.broadcasted_iota(jnp.int32, scores.shape, 1)
    i1 = jnp.argmax(scores, axis=-1)
    s1 = jnp.max(scores, axis=-1)
    masked = jnp.where(lanes == i1[:, None], -jnp.inf, scores)
    i2 = jnp.argmax(masked, axis=-1)
    s2 = jnp.max(masked, axis=-1)
    den = s1 + s2 + 1e-20
    w1_ref[...] = s1 / den
    w2_ref[...] = s2 / den

    oh1 = (lanes == i1[:, None]).astype(jnp.int32)
    oh2 = (lanes == i2[:, None]).astype(jnp.int32)
    oh = oh1 + oh2
    inc = oh
    shift = 1
    while shift < N:
        inc = inc + jnp.pad(inc, ((shift, 0), (0, 0)))[:N, :]
        shift *= 2
    excl = inc - oh
    counts = inc[N - 1:N, :]
    nb = (counts + BLK - 1) // BLK
    nbc = nb
    shift = 1
    while shift < E:
        nbc = nbc + jnp.pad(nbc, ((0, 0), (shift, 0)))[:, :E]
        shift *= 2
    seg = (nbc - nb) * BLK
    pos_base = seg + excl
    pos1_ref[...] = jnp.sum(oh1 * pos_base, axis=1)
    pos2_ref[...] = jnp.sum(oh2 * (pos_base + oh1), axis=1)
    nbc_ref[...] = nbc.reshape(E)

    bb = lax.broadcasted_iota(jnp.int32, (NB, E), 0)
    be = jnp.sum((nbc <= bb).astype(jnp.int32), axis=1)
    be_ref[...] = jnp.clip(be, 0, E - 1)
    total = jnp.sum(jnp.where(
        lax.broadcasted_iota(jnp.int32, (1, E), 1) == E - 1, nbc, 0))
    bx_ref[...] = jnp.minimum(
        lax.broadcasted_iota(jnp.int32, (NB, E), 0)[:, 0], total - 1)


def _gate_call(x, gate_w):
    return pl.pallas_call(
        _gate_body,
        in_specs=[
            pl.BlockSpec((N, H), lambda: (0, 0)),
            pl.BlockSpec((E, H), lambda: (0, 0)),
        ],
        out_specs=[
            pl.BlockSpec((N,), lambda: (0,)),
            pl.BlockSpec((N,), lambda: (0,)),
            pl.BlockSpec((N,), lambda: (0,)),
            pl.BlockSpec((N,), lambda: (0,)),
            pl.BlockSpec((NB,), lambda: (0,)),
            pl.BlockSpec((E,), lambda: (0,)),
            pl.BlockSpec((NB,), lambda: (0,)),
        ],
        out_shape=[
            jax.ShapeDtypeStruct((N,), jnp.int32),
            jax.ShapeDtypeStruct((N,), jnp.int32),
            jax.ShapeDtypeStruct((N,), jnp.float32),
            jax.ShapeDtypeStruct((N,), jnp.float32),
            jax.ShapeDtypeStruct((NB,), jnp.int32),
            jax.ShapeDtypeStruct((E,), jnp.int32),
            jax.ShapeDtypeStruct((NB,), jnp.int32),
        ],
    )(x, gate_w)


def _sc_dispatch(x, pos1, pos2):
    def body(x_hbm, pos1_hbm, pos2_hbm, xs_hbm, idx_v, row_v, sem):
        wid = lax.axis_index("s") * SC_NC + lax.axis_index("c")
        base = wid * TOK_PER_W
        pltpu.sync_copy(x_hbm.at[pl.ds(base, TOK_PER_W)], row_v)
        for pos_hbm in (pos1_hbm, pos2_hbm):
            pltpu.sync_copy(pos_hbm.at[pl.ds(base, TOK_PER_W)], idx_v)
            pltpu.async_copy(row_v, xs_hbm.at[idx_v], sem).wait()

    mesh = plsc.VectorSubcoreMesh(core_axis_name="c", subcore_axis_name="s",
                                  num_cores=SC_NC, num_subcores=SC_NS)
    fn = pl.kernel(body,
                   out_type=jax.ShapeDtypeStruct((PAD, H), jnp.float32),
                   mesh=mesh,
                   scratch_types=[
                       pltpu.VMEM((TOK_PER_W,), jnp.int32),
                       pltpu.VMEM((TOK_PER_W, H), jnp.float32),
                       pltpu.SemaphoreType.DMA,
                   ])
    return fn(x, pos1, pos2)


def _sc_gather(table, pos1, pos2):
    h = int(table.shape[1])

    def body(table_hbm, pos1_hbm, pos2_hbm, out_hbm, iv, rv, sem):
        wid = lax.axis_index("s") * SC_NC + lax.axis_index("c")
        base = wid * TOK_PER_W
        for k, pos_hbm in enumerate((pos1_hbm, pos2_hbm)):
            pltpu.sync_copy(pos_hbm.at[pl.ds(base, TOK_PER_W)], iv)
            pltpu.async_copy(table_hbm.at[iv], rv, sem).wait()
            pltpu.sync_copy(rv, out_hbm.at[pl.ds(k * N + base, TOK_PER_W)])

    mesh = plsc.VectorSubcoreMesh(core_axis_name="c", subcore_axis_name="s",
                                  num_cores=SC_NC, num_subcores=SC_NS)
    fn = pl.kernel(body,
                   out_type=jax.ShapeDtypeStruct((TOPK * N, h), jnp.float32),
                   mesh=mesh,
                   scratch_types=[
                       pltpu.VMEM((TOK_PER_W,), jnp.int32),
                       pltpu.VMEM((TOK_PER_W, h), jnp.float32),
                       pltpu.SemaphoreType.DMA,
                   ])
    return fn(table, pos1, pos2)


def _routed_body(nbc_ref, be_ref, bx_ref, xs_ref, wg_ref, wu_ref, wd_ref,
                 ys_ref):
    i = pl.program_id(0)

    @pl.when(i < nbc_ref[E - 1])
    def _():
        x = xs_ref[...]
        hg = lax.dot_general(x, wg_ref[0], (((1,), (0,)), ((), ())),
                             preferred_element_type=jnp.float32)
        hu = lax.dot_general(x, wu_ref[0], (((1,), (0,)), ((), ())),
                             preferred_element_type=jnp.float32)
        act = (hg * lax.logistic(hg)) * hu
        ys_ref[...] = lax.dot_general(act, wd_ref[0], (((1,), (0,)), ((), ())),
                                      preferred_element_type=jnp.float32)


def _routed_call(nbc, block_expert, bx, xs, w_gate, w_up, w_down):
    grid_spec = pltpu.PrefetchScalarGridSpec(
        num_scalar_prefetch=3,
        grid=(NB,),
        in_specs=[
            pl.BlockSpec((BLK, H), lambda i, t, be, bx: (bx[i], 0)),
            pl.BlockSpec((1, H, DFF), lambda i, t, be, bx: (be[i], 0, 0)),
            pl.BlockSpec((1, H, DFF), lambda i, t, be, bx: (be[i], 0, 0)),
            pl.BlockSpec((1, DFF, H), lambda i, t, be, bx: (be[i], 0, 0)),
        ],
        out_specs=pl.BlockSpec((BLK, H), lambda i, t, be, bx: (bx[i], 0)),
    )
    return pl.pallas_call(
        _routed_body,
        grid_spec=grid_spec,
        out_shape=jax.ShapeDtypeStruct((PAD, H), jnp.float32),
        compiler_params=pltpu.CompilerParams(
            dimension_semantics=("arbitrary",)),
    )(nbc, block_expert, bx, xs, w_gate, w_up, w_down)


def _combine_body(x_ref, swg_ref, swu_ref, swd_ref, y0_ref, y1_ref,
                  w1_ref, w2_ref, y_ref):
    x = x_ref[...]
    hg = lax.dot_general(x, swg_ref[...], (((1,), (0,)), ((), ())),
                         preferred_element_type=jnp.float32)
    hu = lax.dot_general(x, swu_ref[...], (((1,), (0,)), ((), ())),
                         preferred_element_type=jnp.float32)
    act = (hg * lax.logistic(hg)) * hu
    shared = lax.dot_general(act, swd_ref[...], (((1,), (0,)), ((), ())),
                             preferred_element_type=jnp.float32)
    y_ref[...] = shared + w1_ref[...] * y0_ref[...] + w2_ref[...] * y1_ref[...]


def _combine_call(x, sw_gate, sw_up, sw_down, y01, w1, w2):
    bt = 512
    tb = N // bt
    return pl.pallas_call(
        _combine_body,
        grid=(tb,),
        in_specs=[
            pl.BlockSpec((bt, H), lambda t: (t, 0)),
            pl.BlockSpec((H, DFF * NSH), lambda t: (0, 0)),
            pl.BlockSpec((H, DFF * NSH), lambda t: (0, 0)),
            pl.BlockSpec((DFF * NSH, H), lambda t: (0, 0)),
            pl.BlockSpec((bt, H), lambda t: (t, 0)),
            pl.BlockSpec((bt, H), lambda t: (t + tb, 0)),
            pl.BlockSpec((bt, 1), lambda t: (t, 0)),
            pl.BlockSpec((bt, 1), lambda t: (t, 0)),
        ],
        out_specs=pl.BlockSpec((bt, H), lambda t: (t, 0)),
        out_shape=jax.ShapeDtypeStruct((N, H), jnp.float32),
        compiler_params=pltpu.CompilerParams(
            dimension_semantics=("parallel",)),
    )(x, sw_gate, sw_up, sw_down, y01, y01, w1, w2)


def kernel(hidden_states, gate_w, w_gate, w_up, w_down, sw_gate, sw_up, sw_down):
    b, s, h = hidden_states.shape
    x = hidden_states.reshape(-1, h)

    pos1, pos2, w1, w2, block_expert, nbc, bx = _gate_call(x, gate_w)

    xs = _sc_dispatch(x, pos1, pos2)

    ys = _routed_call(nbc, block_expert, bx, xs, w_gate, w_up, w_down)

    y01 = _sc_gather(ys, pos1, pos2)

    y = _combine_call(x, sw_gate, sw_up, sw_down, y01,
                      w1[:, None], w2[:, None])
    return y.reshape(b, s, h)
